# baseline (speedup 1.0000x reference)
import functools

import jax
import jax.numpy as jnp
from jax import lax
from jax.experimental import pallas as pl
from jax.experimental.pallas import tpu as pltpu
from jax.experimental.pallas import tpu_sc as plsc

B = 16384
D = 32
NC = 2
NS = 16
NW = NC * NS
BPW = B // NW
NG = BPW // 16
QS = 17


def _body(u_hbm, i_hbm, f_hbm, ub_hbm, uv_hbm, ib_hbm, iv_hbm, par_hbm,
          out_hbm,
          u_v, i_v, f_v, bu_v, bi_v, vu_v, vi_v, q_v, par_v, out_v, sem):
    wid = lax.axis_index("c") * NS + lax.axis_index("s")
    base = wid * BPW

    pltpu.sync_copy(u_hbm.at[pl.ds(base, BPW)], u_v)
    pltpu.sync_copy(i_hbm.at[pl.ds(base, BPW)], i_v)
    pltpu.sync_copy(f_hbm.at[pl.ds(base, BPW)], f_v)
    pltpu.sync_copy(par_hbm, par_v)

    c1 = pltpu.async_copy(uv_hbm.at[u_v], vu_v, sem)
    c2 = pltpu.async_copy(iv_hbm.at[i_v], vi_v, sem)
    c3 = pltpu.async_copy(ub_hbm.at[u_v], bu_v, sem)
    c4 = pltpu.async_copy(ib_hbm.at[i_v], bi_v, sem)
    c1.wait()
    c2.wait()
    c3.wait()
    c4.wait()

    def body1(b, carry):
        vu0 = vu_v[b, pl.ds(0, 16)]
        vu1 = vu_v[b, pl.ds(16, 16)]
        vi0 = vi_v[b, pl.ds(0, 16)]
        vi1 = vi_v[b, pl.ds(16, 16)]
        q_v[pl.ds(b * QS, 16)] = vu0 * vi0 + vu1 * vi1
        return carry

    lax.fori_loop(0, BPW, body1, 0)

    pv = par_v[pl.ds(0, 16)]
    c0 = pv[0]
    w0 = pv[1]
    w1 = pv[2]
    lane = lax.iota(jnp.int32, 16)

    def body2(g, carry):
        rowbase = g * 16
        idx0 = (rowbase + lane) * QS
        acc = plsc.load_gather(q_v, [idx0])
        for j in range(1, 16):
            acc = acc + plsc.load_gather(q_v, [idx0 + j])
        fv = f_v[pl.ds(rowbase, 16)]
        res = (acc + bu_v[pl.ds(rowbase, 16)] + bi_v[pl.ds(rowbase, 16)]
               + fv * w0 + fv * fv * w1 + c0)
        out_v[pl.ds(rowbase, 16)] = res
        return carry

    lax.fori_loop(0, NG, body2, 0)

    pltpu.sync_copy(out_v, out_hbm.at[pl.ds(base, BPW)])


_mf = functools.partial(
    pl.kernel,
    out_type=jax.ShapeDtypeStruct((B,), jnp.float32),
    mesh=plsc.VectorSubcoreMesh(core_axis_name="c", subcore_axis_name="s"),
    compiler_params=pltpu.CompilerParams(
        needs_layout_passes=False, use_tc_tiling_on_sc=False
    ),
    scratch_types=[
        pltpu.VMEM((BPW,), jnp.int32),
        pltpu.VMEM((BPW,), jnp.int32),
        pltpu.VMEM((BPW,), jnp.float32),
        pltpu.VMEM((BPW,), jnp.float32),
        pltpu.VMEM((BPW,), jnp.float32),
        pltpu.VMEM((BPW, D), jnp.float32),
        pltpu.VMEM((BPW, D), jnp.float32),
        pltpu.VMEM((BPW * QS,), jnp.float32),
        pltpu.VMEM((16,), jnp.float32),
        pltpu.VMEM((BPW,), jnp.float32),
        pltpu.SemaphoreType.DMA,
    ],
)(_body)


def kernel(u, i, f, user_bias, user_vec, item_bias, item_vec, glob_bias,
           frame_w, frame_b):
    params = jnp.concatenate([
        glob_bias + frame_b,
        frame_w.reshape(2),
        jnp.zeros((13,), jnp.float32),
    ])
    return _mf(u.astype(jnp.int32), i.astype(jnp.int32), f,
               user_bias, user_vec, item_bias, item_vec, params)
